# baseline (speedup 1.0000x reference)
import functools
import jax
import jax.numpy as jnp
from jax import lax
from jax.experimental import pallas as pl
from jax.experimental.pallas import tpu as pltpu
from jax.experimental.pallas import tpu_sc as plsc

NUM_TOKENS = 8192
NUM_EXPERTS = 64
CAPACITY = 160
BLK = 256
TPW = 256


def _sc_metadata(x_hbm, e_hbm, w_hbm, xv, ev, wv):
    wid = lax.axis_index("s") * 2 + lax.axis_index("c")
    base = wid * TPW
    pltpu.sync_copy(x_hbm.at[pl.ds(base * NUM_EXPERTS, TPW * NUM_EXPERTS)], xv)
    lane = lax.iota(jnp.int32, 16)

    perms = [
        jnp.bitwise_xor(lane, jnp.int32(sh)) for sh in (8, 4, 2, 1)
    ]

    def _bfly(v, op):
        for p in perms:
            v = op(v, v.at[p].get(mode="promise_in_bounds"))
        return v

    def group(g):
        acc_e = jnp.zeros((16,), jnp.int32)
        acc_w = jnp.zeros((16,), jnp.float32)
        for j in range(16):
            t = g * 16 + j
            off = t * NUM_EXPERTS
            c0 = xv[pl.ds(off, 16)]
            c1 = xv[pl.ds(off + 16, 16)]
            c2 = xv[pl.ds(off + 32, 16)]
            c3 = xv[pl.ds(off + 48, 16)]
            vmax = jnp.maximum(jnp.maximum(c0, c1), jnp.maximum(c2, c3))
            msp = _bfly(vmax, jnp.maximum)
            big = jnp.broadcast_to(jnp.int32(NUM_EXPERTS), (16,))
            cands = [
                jnp.where(c == msp, lane + 16 * k, big)
                for k, c in enumerate((c0, c1, c2, c3))
            ]
            vmin = jnp.minimum(
                jnp.minimum(cands[0], cands[1]), jnp.minimum(cands[2], cands[3])
            )
            esp = _bfly(vmin, jnp.minimum)
            ssum = (
                jnp.exp(c0 - msp)
                + jnp.exp(c1 - msp)
                + jnp.exp(c2 - msp)
                + jnp.exp(c3 - msp)
            )
            wsp = 1.0 / _bfly(ssum, jnp.add)
            sel = lane == j
            acc_e = jnp.where(sel, esp, acc_e)
            acc_w = jnp.where(sel, wsp, acc_w)
        ev[pl.ds(g * 16, 16)] = acc_e
        wv[pl.ds(g * 16, 16)] = acc_w

    pl.loop(0, 16)(group)
    pltpu.sync_copy(ev, e_hbm.at[pl.ds(base, TPW)])
    pltpu.sync_copy(wv, w_hbm.at[pl.ds(base, TPW)])


def _sc_call(x_flat):
    mesh = plsc.VectorSubcoreMesh(core_axis_name="c", subcore_axis_name="s")
    k = functools.partial(
        pl.kernel,
        mesh=mesh,
        out_type=[
            jax.ShapeDtypeStruct((NUM_TOKENS,), jnp.int32),
            jax.ShapeDtypeStruct((NUM_TOKENS,), jnp.float32),
        ],
        scratch_types=[
            pltpu.VMEM((TPW * NUM_EXPERTS,), jnp.float32),
            pltpu.VMEM((TPW,), jnp.int32),
            pltpu.VMEM((TPW,), jnp.float32),
        ],
    )(_sc_metadata)
    return k(x_flat)


def _tc_expand(e_ref, w_ref, combine_ref, aoh_ref, boh_ref, cnt_ref):
    step = pl.program_id(0)

    @pl.when(step == 0)
    def _():
        cnt_ref[...] = jnp.zeros_like(cnt_ref)

    e_row = e_ref[0]
    w_row = w_ref[0]

    e_iota = lax.broadcasted_iota(jnp.int32, (NUM_EXPERTS, BLK), 0)
    oh_msk = e_iota == e_row
    oh = oh_msk.astype(jnp.float32)

    r_iota = lax.broadcasted_iota(jnp.int32, (BLK, BLK), 0)
    c_iota = lax.broadcasted_iota(jnp.int32, (BLK, BLK), 1)
    triu = (r_iota < c_iota).astype(jnp.float32)
    ranks_excl = jnp.dot(oh, triu, preferred_element_type=jnp.float32)

    r_all = cnt_ref[...] + ranks_excl
    r_row = jnp.sum(oh * r_all, axis=0, keepdims=True)
    cnt_ref[...] = cnt_ref[...] + jnp.sum(oh, axis=1, keepdims=True)

    a_mat = oh * w_row
    cap_iota = lax.broadcasted_iota(jnp.int32, (CAPACITY, BLK), 0)
    b_msk = cap_iota == r_row.astype(jnp.int32)
    b_mat = b_msk.astype(jnp.float32)

    combine_ref[...] = a_mat[:, None, :] * b_mat[None, :, :]
    aoh_ref[...] = oh_msk.astype(jnp.int8)
    boh_ref[...] = b_msk.astype(jnp.int8)


def kernel(inputs):
    grid = NUM_TOKENS // BLK
    x_flat = inputs.astype(jnp.float32).reshape(-1)
    e_arr, w_arr = _sc_call(x_flat)
    e3 = e_arr.reshape(grid, 1, BLK)
    w3 = w_arr.reshape(grid, 1, BLK)
    combine_t, aoh_t, boh_t = pl.pallas_call(
        _tc_expand,
        grid=(grid,),
        in_specs=[
            pl.BlockSpec((1, 1, BLK), lambda i: (i, 0, 0)),
            pl.BlockSpec((1, 1, BLK), lambda i: (i, 0, 0)),
        ],
        out_specs=[
            pl.BlockSpec((NUM_EXPERTS, CAPACITY, BLK), lambda i: (0, 0, i)),
            pl.BlockSpec((NUM_EXPERTS, BLK), lambda i: (0, i)),
            pl.BlockSpec((CAPACITY, BLK), lambda i: (0, i)),
        ],
        out_shape=[
            jax.ShapeDtypeStruct((NUM_EXPERTS, CAPACITY, NUM_TOKENS), jnp.float32),
            jax.ShapeDtypeStruct((NUM_EXPERTS, NUM_TOKENS), jnp.int8),
            jax.ShapeDtypeStruct((CAPACITY, NUM_TOKENS), jnp.int8),
        ],
        scratch_shapes=[pltpu.VMEM((NUM_EXPERTS, 1), jnp.float32)],
    )(e3, w3)
    combine = jnp.transpose(combine_t, (2, 0, 1))
    a_bool = jnp.transpose(aoh_t, (1, 0)).view(jnp.bool_)
    b_bool = jnp.transpose(boh_t, (1, 0)).view(jnp.bool_)
    sec = a_bool[:, :, None] & b_bool[:, None, :]
    return (combine, sec)


# speedup vs baseline: 1.1770x; 1.1770x over previous
import jax
import jax.numpy as jnp
from jax import lax
from jax.experimental import pallas as pl
from jax.experimental.pallas import tpu as pltpu

NUM_TOKENS = 8192
NUM_EXPERTS = 64
CAPACITY = 160
BLK = 256


def _router_kernel(x_ref, combine_ref, aoh_ref, boh_ref, cnt_ref):
    step = pl.program_id(0)

    @pl.when(step == 0)
    def _():
        cnt_ref[...] = jnp.zeros_like(cnt_ref)

    x = x_ref[...]
    m = jnp.max(x, axis=0, keepdims=True)
    s = jnp.sum(jnp.exp(x - m), axis=0, keepdims=True)
    w_row = 1.0 / s

    e_iota = lax.broadcasted_iota(jnp.int32, (NUM_EXPERTS, BLK), 0)
    cand = jnp.where(x == m, e_iota, NUM_EXPERTS)
    e_row = jnp.min(cand, axis=0, keepdims=True)

    oh_msk = e_iota == e_row
    oh = oh_msk.astype(jnp.float32)

    r_iota = lax.broadcasted_iota(jnp.int32, (BLK, BLK), 0)
    c_iota = lax.broadcasted_iota(jnp.int32, (BLK, BLK), 1)
    triu = (r_iota < c_iota).astype(jnp.float32)
    ranks_excl = jnp.dot(oh, triu, preferred_element_type=jnp.float32)

    r_all = cnt_ref[...] + ranks_excl
    r_row = jnp.sum(oh * r_all, axis=0, keepdims=True)
    cnt_ref[...] = cnt_ref[...] + jnp.sum(oh, axis=1, keepdims=True)

    a_mat = oh * w_row
    cap_iota = lax.broadcasted_iota(jnp.int32, (CAPACITY, BLK), 0)
    b_msk = cap_iota == r_row.astype(jnp.int32)
    b_mat = b_msk.astype(jnp.float32)

    combine_ref[...] = a_mat[:, None, :] * b_mat[None, :, :]
    aoh_ref[...] = oh_msk.astype(jnp.int8)
    boh_ref[...] = b_msk.astype(jnp.int8)


def kernel(inputs):
    grid = NUM_TOKENS // BLK
    x_t = inputs.astype(jnp.float32).T
    combine_t, aoh_t, boh_t = pl.pallas_call(
        _router_kernel,
        grid=(grid,),
        in_specs=[pl.BlockSpec((NUM_EXPERTS, BLK), lambda i: (0, i))],
        out_specs=[
            pl.BlockSpec((NUM_EXPERTS, CAPACITY, BLK), lambda i: (0, 0, i)),
            pl.BlockSpec((NUM_EXPERTS, BLK), lambda i: (0, i)),
            pl.BlockSpec((CAPACITY, BLK), lambda i: (0, i)),
        ],
        out_shape=[
            jax.ShapeDtypeStruct((NUM_EXPERTS, CAPACITY, NUM_TOKENS), jnp.float32),
            jax.ShapeDtypeStruct((NUM_EXPERTS, NUM_TOKENS), jnp.int8),
            jax.ShapeDtypeStruct((CAPACITY, NUM_TOKENS), jnp.int8),
        ],
        scratch_shapes=[pltpu.VMEM((NUM_EXPERTS, 1), jnp.float32)],
    )(x_t)
    combine = jnp.transpose(combine_t, (2, 0, 1))
    a_bool = jnp.transpose(aoh_t, (1, 0)).view(jnp.bool_)
    b_bool = jnp.transpose(boh_t, (1, 0)).view(jnp.bool_)
    sec = a_bool[:, :, None] & b_bool[:, None, :]
    return (combine, sec)
